# Initial kernel scaffold; baseline (speedup 1.0000x reference)
import jax
import jax.numpy as jnp
from jax.experimental import pallas as pl


def kernel(indices, table):
    raise NotImplementedError("write your pallas kernel here")


# speedup vs baseline: 4.0888x; 4.0888x over previous
import functools

import jax
import jax.numpy as jnp
from jax import lax
from jax.experimental import pallas as pl
from jax.experimental.pallas import tpu as pltpu
from jax.experimental.pallas import tpu_sc as plsc

NUM_SETS = 4
SET_LEN = 64
EMBED_DIM = 64
BATCH = 16384
B_FLAT = BATCH * NUM_SETS

NC = 2
NS = 16
L = 16
NW = NC * NS
BPW = B_FLAT // NW
CH = 128
NCH = BPW // CH

_mesh = plsc.VectorSubcoreMesh(core_axis_name="c", subcore_axis_name="s")


@functools.partial(
    pl.kernel,
    mesh=_mesh,
    out_type=jax.ShapeDtypeStruct((B_FLAT, EMBED_DIM), jnp.float32),
    scratch_types=[
        pltpu.VMEM((NCH, CH), jnp.int32),
        pltpu.VMEM((CH, EMBED_DIM), jnp.float32),
        pltpu.VMEM((CH, EMBED_DIM), jnp.float32),
        pltpu.SemaphoreType.DMA,
        pltpu.SemaphoreType.DMA,
    ],
    compiler_params=pltpu.CompilerParams(use_tc_tiling_on_sc=False),
)
def _sc_gather(idx_hbm, table_hbm, out_hbm, idx_v, rows0, rows1, sem0, sem1):
    wid = lax.axis_index("s") * NC + lax.axis_index("c")
    base = wid * BPW

    pltpu.sync_copy(idx_hbm.at[pl.ds(wid * NCH, NCH)], idx_v)

    offv = (lax.iota(jnp.int32, L) % NUM_SETS) * SET_LEN
    for c in range(NCH):
        row = idx_v.at[c]
        for i in range(CH // L):
            row[pl.ds(i * L, L)] = row[pl.ds(i * L, L)] + offv

    bufs = (rows0, rows1)
    sems = (sem0, sem1)

    def gstart(c):
        return pltpu.async_copy(table_hbm.at[idx_v.at[c]], bufs[c % 2], sems[c % 2])

    cp = gstart(0)
    for c in range(NCH):
        nxt = gstart(c + 1) if c + 1 < NCH else None
        cp.wait()
        pltpu.sync_copy(bufs[c % 2], out_hbm.at[pl.ds(base + c * CH, CH)])
        cp = nxt


def kernel(indices, table):
    idx2d = indices.astype(jnp.int32).reshape(NW * NCH, CH)
    out = _sc_gather(idx2d, table)
    return out.reshape(BATCH, NUM_SETS, EMBED_DIM)


# speedup vs baseline: 5.3741x; 1.3143x over previous
import functools

import jax
import jax.numpy as jnp
from jax import lax
from jax.experimental import pallas as pl
from jax.experimental.pallas import tpu as pltpu
from jax.experimental.pallas import tpu_sc as plsc

NUM_SETS = 4
SET_LEN = 64
EMBED_DIM = 64
BATCH = 16384
TBL_ROWS = NUM_SETS * SET_LEN
TBL_WORDS = TBL_ROWS * EMBED_DIM

NC = 2
NS = 16
L = 16
NW = NC * NS
NBT = BATCH // 128
TB = 32
NCHK = NBT // TB

_mesh = plsc.VectorSubcoreMesh(core_axis_name="c", subcore_axis_name="s")


@functools.partial(
    pl.kernel,
    mesh=_mesh,
    out_type=jax.ShapeDtypeStruct((NW, NBT, 8, 128), jnp.float32),
    scratch_types=[
        pltpu.VMEM((TBL_WORDS,), jnp.float32),
        pltpu.VMEM((NBT, 1, 128), jnp.int32),
        pltpu.VMEM((TB, 8, 128), jnp.float32),
        pltpu.VMEM((TB, 8, 128), jnp.float32),
        pltpu.SemaphoreType.DMA,
        pltpu.SemaphoreType.DMA,
    ],
    compiler_params=pltpu.CompilerParams(
        use_tc_tiling_on_sc=False, needs_layout_passes=False
    ),
)
def _sc_gather(tbl_hbm, idx_hbm, out_hbm, tbl_v, idx_v, buf0, buf1, sem0, sem1):
    w = lax.axis_index("s") * NC + lax.axis_index("c")
    j = w // 8
    ct = w % 8
    base = j * (SET_LEN * EMBED_DIM) + ct * 8

    pltpu.sync_copy(tbl_hbm, tbl_v)
    pltpu.sync_copy(idx_hbm.at[pl.ds(0, NBT), pl.ds(j, 1), pl.ds(0, 128)], idx_v)

    bufs = (buf0, buf1)
    sems = (sem0, sem1)
    cps = [None, None]
    for chunk in range(NCHK):
        b = chunk % 2
        buf = bufs[b]
        if cps[b] is not None:
            cps[b].wait()

        def body(t, carry, _chunk=chunk, _buf=buf):
            bt = _chunk * TB + t
            row = idx_v.at[bt, 0]
            for r8 in range(8):
                im = row[pl.ds(r8 * L, L)] * EMBED_DIM
                for ci in range(8):
                    g = plsc.load_gather(tbl_v, [im + (base + ci)])
                    dst = _buf.at[t, ci]
                    dst[pl.ds(r8 * L, L)] = g
            return carry

        lax.fori_loop(0, TB, body, 0)
        cps[b] = pltpu.async_copy(
            buf, out_hbm.at[w, pl.ds(chunk * TB, TB)], sems[b]
        )
    for cp in cps:
        if cp is not None:
            cp.wait()


def kernel(indices, table):
    idx3 = jnp.transpose(
        jnp.transpose(indices.astype(jnp.int32)).reshape(NUM_SETS, NBT, 128),
        (1, 0, 2),
    )
    tbl_flat = jax.lax.slice(table, (0, 0), (TBL_ROWS, EMBED_DIM)).reshape(-1)
    out4 = _sc_gather(tbl_flat, idx3)
    z = out4.reshape(NUM_SETS, 8, NBT, 8, 128)
    return jnp.transpose(z, (2, 4, 0, 1, 3)).reshape(BATCH, NUM_SETS, EMBED_DIM)
